# baseline (speedup 1.0000x reference)
import functools

import jax
import jax.numpy as jnp
from jax import lax
from jax.experimental import pallas as pl
from jax.experimental.pallas import tpu as pltpu
from jax.experimental.pallas import tpu_sc as plsc

_B, _C, _K, _D = 4096, 20, 20, 64
_NC, _NS = 2, 16
_NW = _NC * _NS
_EPW = _B // _NW
_E = 16
_NCHUNK = _EPW // _E
_P = _E * _C
_GSUB = 4
_SUB = _P // _GSUB


def _sc_pred(ctx_idx, foc_idx, cemb, nemb):
    mesh = plsc.VectorSubcoreMesh(core_axis_name="c", subcore_axis_name="s")

    @functools.partial(
        pl.kernel,
        out_type=jax.ShapeDtypeStruct((_B * _K,), jnp.float32),
        mesh=mesh,
        scratch_types=[
            pltpu.VMEM((_P,), jnp.int32),
            pltpu.VMEM((_P,), jnp.int32),
            pltpu.VMEM((_P, _D), jnp.float32),
            pltpu.VMEM((_P, _D), jnp.float32),
            pltpu.VMEM((_E, _D), jnp.float32),
            pltpu.VMEM((_P,), jnp.float32),
            pltpu.SemaphoreType.DMA,
            pltpu.SemaphoreType.DMA,
        ],
        compiler_params=pltpu.CompilerParams(
            use_tc_tiling_on_sc=False, needs_layout_passes=False),
    )
    def k(ci_hbm, fi_hbm, ce_hbm, ne_hbm, pred_hbm,
          ci_v, fi_v, cr_v, tr_v, src_v, pr_v, sem1, sem2):
        wid = lax.axis_index("s") * _NC + lax.axis_index("c")
        lane = lax.iota(jnp.int32, 16)

        def chunk_body(c, carry):
            po = (wid * _EPW + c * _E) * _C
            pltpu.sync_copy(ci_hbm.at[pl.ds(po, _P)], ci_v)
            pltpu.sync_copy(fi_hbm.at[pl.ds(po, _P)], fi_v)
            copies = []
            for i in range(_GSUB):
                s = pl.ds(i * _SUB, _SUB)
                copies.append(
                    pltpu.async_copy(ce_hbm.at[ci_v.at[s]], cr_v.at[s], sem1))
                copies.append(
                    pltpu.async_copy(ne_hbm.at[fi_v.at[s]], tr_v.at[s], sem2))
            for cp in copies:
                cp.wait()

            def ebody(e, ecarry):
                base = e * _C
                for d4 in range(_D // 16):
                    sl = pl.ds(d4 * 16, 16)
                    acc = cr_v[base, sl]
                    for cc in range(1, _C):
                        acc = acc + cr_v[base + cc, sl]
                    src_v[e, sl] = acc
                return ecarry
            lax.fori_loop(0, _E, ebody, 0)

            def gbody(g, gcarry):
                row = g * 16 + lane
                b_loc = row // _K
                acc = jnp.zeros((16,), jnp.float32)
                for d in range(_D):
                    dsp = jnp.full((16,), d, jnp.int32)
                    s = plsc.load_gather(src_v, [b_loc, dsp])
                    t = plsc.load_gather(tr_v, [row, dsp])
                    acc = acc + s * t
                pr_v[pl.ds(g * 16, 16)] = acc
                return gcarry
            lax.fori_loop(0, _P // 16, gbody, 0)

            pltpu.sync_copy(pr_v, pred_hbm.at[pl.ds(po, _P)])
            return carry

        lax.fori_loop(0, _NCHUNK, chunk_body, 0)

    return k(ctx_idx, foc_idx, cemb, nemb)


def _tc_loss_body(p_ref, w_ref, l_ref, o_ref):
    p = p_ref[...]
    w = w_ref[...]
    lbl = l_ref[...]
    bce = jnp.maximum(p, 0.0) - p * lbl + jnp.log1p(jnp.exp(-jnp.abs(p)))
    num = jnp.sum(w * bce, axis=1, keepdims=True)
    den = jnp.sum(w, axis=1, keepdims=True)
    o_ref[...] = jnp.sum(num / den, axis=0, keepdims=True) / p_ref.shape[0]


def kernel(input, focus_word, weight_mask, labels, context_emb, neg_emb):
    ci = input.reshape(-1)
    fi = focus_word.reshape(-1)
    pred = _sc_pred(ci, fi, context_emb, neg_emb)
    loss = pl.pallas_call(
        _tc_loss_body,
        out_shape=jax.ShapeDtypeStruct((1, 1), jnp.float32),
    )(pred.reshape(_B, _K), weight_mask, labels)
    return loss[0, 0]
